# Initial kernel scaffold; baseline (speedup 1.0000x reference)
import functools

import numpy as np
import jax
import jax.numpy as jnp
from jax import lax
from jax.experimental import pallas as pl
from jax.experimental.pallas import tpu as pltpu
from jax.experimental.pallas import tpu_sc as plsc

_VOCAB = 1_000_000
_B = 4096
_S = 200
_LR = 0.01
_DECAY = 0.99
_W_MIN = 0.0
_W_MAX = 1.0
_TAU = 5.0

_NC = 2
_NS = 16
_NW = _NC * _NS
_TOTAL = _B * _S
_PER_W = _TOTAL // _NW
_CHUNK = 128
_NCHUNK = _PER_W // _CHUNK
_VPAD = 1 << 20
_SLICE = _VPAD // _NS
_ZBUF = 8192


def _decay_matrix() -> np.ndarray:
    d = float(np.exp(np.float32(-1.0 / np.float32(_TAU))))
    k = np.arange(_S)
    e = k[None, :] - k[:, None]
    m = np.where(e >= 0, np.power(d, np.maximum(e, 0)), 0.0)
    return m.astype(np.float32)


_M_CONST = _decay_matrix()


_A_ROWS = 1024


def _updates_body(s_ref, m_ref, o_ref):
    s = s_ref[...]
    t = lax.dot_general(
        s, m_ref[...], (((1,), (0,)), ((), ())),
        precision=lax.Precision.HIGHEST,
        preferred_element_type=jnp.float32,
    )
    o_ref[...] = (_LR * t) * s


_updates_call = pl.pallas_call(
    _updates_body,
    grid=(_B // _A_ROWS,),
    in_specs=[
        pl.BlockSpec((_A_ROWS, _S), lambda i: (i, 0)),
        pl.BlockSpec((_S, _S), lambda i: (0, 0)),
    ],
    out_specs=pl.BlockSpec((_A_ROWS, _S), lambda i: (i, 0)),
    out_shape=jax.ShapeDtypeStruct((_B, _S), jnp.float32),
)


_mesh = plsc.VectorSubcoreMesh(core_axis_name="c", subcore_axis_name="s")


@functools.partial(
    pl.kernel,
    out_type=jax.ShapeDtypeStruct((_NC, _VPAD), jnp.float32),
    mesh=_mesh,
    scratch_types=[
        pltpu.VMEM((_NCHUNK, _CHUNK), jnp.int32),
        pltpu.VMEM((_NCHUNK, _CHUNK), jnp.float32),
        pltpu.VMEM((_ZBUF,), jnp.float32),
        pltpu.VMEM_SHARED((_VPAD,), jnp.float32),
        pltpu.SemaphoreType.DMA,
        pltpu.SemaphoreType.DMA,
    ],
)
def _scatter_kernel(ids_hbm, upd_hbm, out_hbm, idx_v, val_v, zbuf, accum,
                    sem_i, sem_v):
    c = lax.axis_index("c")
    s = lax.axis_index("s")
    wid = c * _NS + s
    cp_i = pltpu.async_copy(ids_hbm.at[wid], idx_v, sem_i)
    cp_v = pltpu.async_copy(upd_hbm.at[wid], val_v, sem_v)

    zero = jnp.zeros((16,), jnp.float32)

    def zfill(i, carry):
        zbuf[pl.ds(i * 16, 16)] = zero
        return carry

    lax.fori_loop(0, _ZBUF // 16, zfill, 0)

    def zdma(i, carry):
        pltpu.sync_copy(zbuf, accum.at[pl.ds(s * _SLICE + i * _ZBUF, _ZBUF)])
        return carry

    lax.fori_loop(0, _SLICE // _ZBUF, zdma, 0)
    plsc.subcore_barrier()

    cp_i.wait()
    cp_v.wait()

    def body(j, carry):
        pltpu.sync_copy(val_v.at[j], accum.at[idx_v.at[j]], add=True)
        return carry

    lax.fori_loop(0, _NCHUNK, body, 0)
    plsc.subcore_barrier()

    pltpu.sync_copy(accum.at[pl.ds(s * _SLICE, _SLICE)],
                    out_hbm.at[c, pl.ds(s * _SLICE, _SLICE)])


_C_COLS = 8000
_C_GRID = _VOCAB // _C_COLS


def _combine_body(p_ref, w_ref, o_ref):
    acc = w_ref[...] + p_ref[0, :] + p_ref[1, :]
    o_ref[...] = jnp.clip(acc * _DECAY, _W_MIN, _W_MAX)


_combine_call = pl.pallas_call(
    _combine_body,
    grid=(_C_GRID,),
    in_specs=[
        pl.BlockSpec((_NC, _C_COLS), lambda i: (0, i)),
        pl.BlockSpec((_C_COLS,), lambda i: (i,)),
    ],
    out_specs=pl.BlockSpec((_C_COLS,), lambda i: (i,)),
    out_shape=jax.ShapeDtypeStruct((_VOCAB,), jnp.float32),
)


def kernel(token_ids, spikes, token_weights):
    updates = _updates_call(spikes, jnp.asarray(_M_CONST))
    ids3 = token_ids.reshape(_NW, _NCHUNK, _CHUNK)
    upd3 = updates.reshape(_NW, _NCHUNK, _CHUNK)
    partials = _scatter_kernel(ids3, upd3)
    return _combine_call(partials, token_weights)


# speedup vs baseline: 22.1498x; 22.1498x over previous
import functools

import numpy as np
import jax
import jax.numpy as jnp
from jax import lax
from jax.experimental import pallas as pl
from jax.experimental.pallas import tpu as pltpu
from jax.experimental.pallas import tpu_sc as plsc

_VOCAB = 1_000_000
_B = 4096
_S = 200
_LR = 0.01
_DECAY = 0.99
_W_MIN = 0.0
_W_MAX = 1.0
_TAU = 5.0

_NC = 2
_NS = 16
_NW = _NC * _NS
_TOTAL = _B * _S
_PER_W = _TOTAL // _NW
_CHUNK = 128
_NCHUNK = _PER_W // _CHUNK
_VPAD = 1 << 20
_SLICE = _VPAD // _NS
_TAIL = _VOCAB - (_NS - 1) * _SLICE
_ZBUF = 8192


def _decay_matrix() -> np.ndarray:
    d = float(np.exp(np.float32(-1.0 / np.float32(_TAU))))
    k = np.arange(_S)
    e = k[None, :] - k[:, None]
    m = np.where(e >= 0, np.power(d, np.maximum(e, 0)), 0.0)
    return m.astype(np.float32)


_M_CONST = _decay_matrix()


_A_ROWS = 1024


def _updates_body(s_ref, m_ref, o_ref):
    s = s_ref[...]
    t = lax.dot_general(
        s, m_ref[...], (((1,), (0,)), ((), ())),
        precision=lax.Precision.HIGHEST,
        preferred_element_type=jnp.float32,
    )
    o_ref[...] = (_LR * t) * s


_updates_call = pl.pallas_call(
    _updates_body,
    grid=(_B // _A_ROWS,),
    in_specs=[
        pl.BlockSpec((_A_ROWS, _S), lambda i: (i, 0)),
        pl.BlockSpec((_S, _S), lambda i: (0, 0)),
    ],
    out_specs=pl.BlockSpec((_A_ROWS, _S), lambda i: (i, 0)),
    out_shape=jax.ShapeDtypeStruct((_B, _S), jnp.float32),
)


_mesh = plsc.VectorSubcoreMesh(core_axis_name="c", subcore_axis_name="s")


@functools.partial(
    pl.kernel,
    out_type=(jax.ShapeDtypeStruct((_VPAD,), jnp.float32),
              jax.ShapeDtypeStruct((_VPAD,), jnp.float32)),
    mesh=_mesh,
    scratch_types=[
        pltpu.VMEM((_NCHUNK, _CHUNK), jnp.int32),
        pltpu.VMEM((_NCHUNK, _CHUNK), jnp.float32),
        pltpu.VMEM((_ZBUF,), jnp.float32),
        pltpu.VMEM_SHARED((_VPAD,), jnp.float32),
        pltpu.SemaphoreType.DMA,
        pltpu.SemaphoreType.DMA,
    ],
)
def _scatter_kernel(ids_hbm, upd_hbm, out0_hbm, out1_hbm, idx_v, val_v, zbuf,
                    accum, sem_i, sem_v):
    c = lax.axis_index("c")
    s = lax.axis_index("s")
    wid = c * _NS + s
    cp_i = pltpu.async_copy(ids_hbm.at[wid], idx_v, sem_i)
    cp_v = pltpu.async_copy(upd_hbm.at[wid], val_v, sem_v)

    zero = jnp.zeros((16,), jnp.float32)

    def zfill(i, carry):
        zbuf[pl.ds(i * 16, 16)] = zero
        return carry

    lax.fori_loop(0, _ZBUF // 16, zfill, 0)

    def zdma(i, carry):
        pltpu.sync_copy(zbuf, accum.at[pl.ds(s * _SLICE + i * _ZBUF, _ZBUF)])
        return carry

    lax.fori_loop(0, _SLICE // _ZBUF, zdma, 0)
    plsc.subcore_barrier()

    cp_i.wait()
    cp_v.wait()

    def body(j, carry):
        pltpu.sync_copy(val_v.at[j], accum.at[idx_v.at[j]], add=True)
        return carry

    lax.fori_loop(0, _NCHUNK, body, 0)
    plsc.subcore_barrier()

    @pl.when(c == 0)
    def _to0():
        pltpu.sync_copy(accum.at[pl.ds(s * _SLICE, _SLICE)],
                        out0_hbm.at[pl.ds(s * _SLICE, _SLICE)])

    @pl.when(c == 1)
    def _to1():
        pltpu.sync_copy(accum.at[pl.ds(s * _SLICE, _SLICE)],
                        out1_hbm.at[pl.ds(s * _SLICE, _SLICE)])


_C_ROWS = 128
_C_COLS = _VPAD // _C_ROWS
_C_BLK = 8
_C_GRID = _C_ROWS // _C_BLK


def _combine_body(p0_ref, p1_ref, w_ref, o_ref):
    acc = w_ref[...] + p0_ref[...] + p1_ref[...]
    o_ref[...] = jnp.clip(acc * _DECAY, _W_MIN, _W_MAX)


_combine_call = pl.pallas_call(
    _combine_body,
    grid=(_C_GRID,),
    in_specs=[
        pl.BlockSpec((_C_BLK, _C_COLS), lambda i: (i, 0)),
        pl.BlockSpec((_C_BLK, _C_COLS), lambda i: (i, 0)),
        pl.BlockSpec((_C_BLK, _C_COLS), lambda i: (i, 0)),
    ],
    out_specs=pl.BlockSpec((_C_BLK, _C_COLS), lambda i: (i, 0)),
    out_shape=jax.ShapeDtypeStruct((_C_ROWS, _C_COLS), jnp.float32),
)


def kernel(token_ids, spikes, token_weights):
    updates = _updates_call(spikes, jnp.asarray(_M_CONST))
    ids3 = token_ids.reshape(_NW, _NCHUNK, _CHUNK)
    upd3 = updates.reshape(_NW, _NCHUNK, _CHUNK)
    p0, p1 = _scatter_kernel(ids3, upd3)
    p0 = p0.reshape(_C_ROWS, _C_COLS)
    p1 = p1.reshape(_C_ROWS, _C_COLS)
    tw2 = jnp.pad(token_weights, (0, _VPAD - _VOCAB)).reshape(_C_ROWS, _C_COLS)
    w = _combine_call(p0, p1, tw2).reshape(_VPAD)
    return w[:_VOCAB]
